# baseline (speedup 1.0000x reference)
import functools

import jax
import jax.numpy as jnp
from jax import lax
from jax.experimental import pallas as pl
from jax.experimental.pallas import tpu as pltpu
from jax.experimental.pallas import tpu_sc as plsc

BATCH = 16384
HIST = 200
N = BATCH * HIST
VOCAB_N = 1_000_000
ACTIVE_S = 8
NUM_WORKERS = 2 * ACTIVE_S
BPW = N // NUM_WORKERS
CHUNK = 12_800
NCHUNK = BPW // CHUNK
SEG = 10_000
NSEG = VOCAB_N // SEG


def _make_kernel():
    mesh = plsc.VectorSubcoreMesh(core_axis_name="c", subcore_axis_name="s")

    @functools.partial(
        pl.kernel,
        mesh=mesh,
        out_type=jax.ShapeDtypeStruct((N,), jnp.float32),
        scratch_types=[
            pltpu.VMEM_SHARED((VOCAB_N,), jnp.float32),
            pltpu.VMEM((CHUNK,), jnp.int32),
            pltpu.VMEM((CHUNK,), jnp.int32),
            pltpu.VMEM((CHUNK,), jnp.float32),
            pltpu.VMEM((CHUNK,), jnp.float32),
            pltpu.SemaphoreType.DMA,
            pltpu.SemaphoreType.DMA,
            pltpu.SemaphoreType.DMA,
            pltpu.SemaphoreType.DMA,
            pltpu.SemaphoreType.DMA,
        ],
    )
    def gather_kernel(tok_hbm, tab_hbm, out_hbm, tab_sp, idx0, idx1,
                      val0, val1, si0, si1, so0, so1, sg):
        s = lax.axis_index("s")
        wid = s * 2 + lax.axis_index("c")
        base = wid * BPW
        idx = (idx0, idx1)
        val = (val0, val1)
        sem_i = (si0, si1)
        sem_o = (so0, so1)

        @pl.when(s < ACTIVE_S)
        def _():
            pltpu.async_copy(tok_hbm.at[pl.ds(base, CHUNK)], idx0, si0)

        for r in range((NSEG + 15) // 16):

            @pl.when(r * 16 + s < NSEG)
            def _():
                toff = (r * 16 + s) * SEG
                pltpu.sync_copy(tab_hbm.at[pl.ds(toff, SEG)],
                                val0.at[pl.ds(0, SEG)])
                pltpu.sync_copy(val0.at[pl.ds(0, SEG)],
                                tab_sp.at[pl.ds(toff, SEG)])

        plsc.subcore_barrier()

        @pl.when(s < ACTIVE_S)
        def _():
            for i in range(NCHUNK):
                b = i % 2
                if i + 1 < NCHUNK:
                    pltpu.async_copy(
                        tok_hbm.at[pl.ds(base + (i + 1) * CHUNK, CHUNK)],
                        idx[1 - b], sem_i[1 - b])
                if i >= 2:
                    pltpu.make_async_copy(
                        val[b],
                        out_hbm.at[pl.ds(base + (i - 2) * CHUNK, CHUNK)],
                        sem_o[b]).wait()
                pltpu.make_async_copy(
                    tok_hbm.at[pl.ds(base + i * CHUNK, CHUNK)], idx[b],
                    sem_i[b]).wait()
                pltpu.async_copy(tab_sp.at[idx[b]], val[b], sg).wait()
                pltpu.async_copy(
                    val[b], out_hbm.at[pl.ds(base + i * CHUNK, CHUNK)],
                    sem_o[b])
            for i in range(NCHUNK - 2, NCHUNK):
                b = i % 2
                pltpu.make_async_copy(
                    val[b], out_hbm.at[pl.ds(base + i * CHUNK, CHUNK)],
                    sem_o[b]).wait()

    return gather_kernel


_GATHER = _make_kernel()


def kernel(tokens, vocab_table):
    flat = tokens.reshape(N)
    out = _GATHER(flat, vocab_table)
    return out.reshape(BATCH, HIST)


# speedup vs baseline: 1.0944x; 1.0944x over previous
import functools

import jax
import jax.numpy as jnp
from jax import lax
from jax.experimental import pallas as pl
from jax.experimental.pallas import tpu as pltpu
from jax.experimental.pallas import tpu_sc as plsc

BATCH = 16384
HIST = 200
N = BATCH * HIST
VOCAB_N = 1_000_000
CHUNK = 12_800
SEG = 10_000
NSEG = VOCAB_N // SEG

NC_HBM = 3
NC_SP_HI = 10
NC_SP_LO = 9


def _make_kernel():
    mesh = plsc.VectorSubcoreMesh(core_axis_name="c", subcore_axis_name="s")

    @functools.partial(
        pl.kernel,
        mesh=mesh,
        out_type=jax.ShapeDtypeStruct((N,), jnp.float32),
        scratch_types=[
            pltpu.VMEM_SHARED((VOCAB_N,), jnp.float32),
            pltpu.VMEM((CHUNK,), jnp.int32),
            pltpu.VMEM((CHUNK,), jnp.int32),
            pltpu.VMEM((CHUNK,), jnp.float32),
            pltpu.VMEM((CHUNK,), jnp.float32),
            pltpu.SemaphoreType.DMA,
            pltpu.SemaphoreType.DMA,
            pltpu.SemaphoreType.DMA,
            pltpu.SemaphoreType.DMA,
            pltpu.SemaphoreType.DMA,
        ],
    )
    def gather_kernel(tok_hbm, tab_hbm, out_hbm, tab_sp, idx0, idx1,
                      val0, val1, si0, si1, so0, so1, sg):
        s = lax.axis_index("s")
        c = lax.axis_index("c")
        idx = (idx0, idx1)
        val = (val0, val1)
        sem_i = (si0, si1)
        sem_o = (so0, so1)

        nck = jnp.where(s < 4, NC_HBM,
                        jnp.where(s < 12, NC_SP_HI, NC_SP_LO))
        pre = 2 * (NC_HBM * jnp.minimum(s, 4)
                   + NC_SP_HI * jnp.clip(s - 4, 0, 8)
                   + NC_SP_LO * jnp.maximum(s - 12, 0)) + c * nck
        base = pre * CHUNK

        pltpu.async_copy(tok_hbm.at[pl.ds(base, CHUNK)], idx0, si0)

        for r in range((NSEG + 15) // 16):

            @pl.when(r * 16 + s < NSEG)
            def _():
                toff = (r * 16 + s) * SEG
                pltpu.sync_copy(tab_hbm.at[pl.ds(toff, SEG)],
                                val0.at[pl.ds(0, SEG)])
                pltpu.sync_copy(val0.at[pl.ds(0, SEG)],
                                tab_sp.at[pl.ds(toff, SEG)])

        plsc.subcore_barrier()

        def run_loop(nchunk, src):
            for i in range(nchunk):
                b = i % 2
                if i + 1 < nchunk:
                    pltpu.async_copy(
                        tok_hbm.at[pl.ds(base + (i + 1) * CHUNK, CHUNK)],
                        idx[1 - b], sem_i[1 - b])
                if i >= 2:
                    pltpu.make_async_copy(
                        val[b],
                        out_hbm.at[pl.ds(base + (i - 2) * CHUNK, CHUNK)],
                        sem_o[b]).wait()
                pltpu.make_async_copy(
                    tok_hbm.at[pl.ds(base + i * CHUNK, CHUNK)], idx[b],
                    sem_i[b]).wait()
                pltpu.async_copy(src.at[idx[b]], val[b], sg).wait()
                pltpu.async_copy(
                    val[b], out_hbm.at[pl.ds(base + i * CHUNK, CHUNK)],
                    sem_o[b])
            for i in range(max(0, nchunk - 2), nchunk):
                b = i % 2
                pltpu.make_async_copy(
                    val[b], out_hbm.at[pl.ds(base + i * CHUNK, CHUNK)],
                    sem_o[b]).wait()

        @pl.when(s < 4)
        def _():
            run_loop(NC_HBM, tab_hbm)

        @pl.when(jnp.logical_and(s >= 4, s < 12))
        def _():
            run_loop(NC_SP_HI, tab_sp)

        @pl.when(s >= 12)
        def _():
            run_loop(NC_SP_LO, tab_sp)

    return gather_kernel


_GATHER = _make_kernel()


def kernel(tokens, vocab_table):
    flat = tokens.reshape(N)
    out = _GATHER(flat, vocab_table)
    return out.reshape(BATCH, HIST)


# speedup vs baseline: 1.2281x; 1.1221x over previous
import functools

import jax
import jax.numpy as jnp
from jax import lax
from jax.experimental import pallas as pl
from jax.experimental.pallas import tpu as pltpu
from jax.experimental.pallas import tpu_sc as plsc

BATCH = 16384
HIST = 200
N = BATCH * HIST
VOCAB_N = 1_000_000
NUM_WORKERS = 32
BPW = N // NUM_WORKERS
CHUNK = 12_800
NCHUNK = BPW // CHUNK
SEG = 10_000
NSEG = VOCAB_N // SEG
NROUND = (NSEG + 15) // 16


def _make_kernel():
    mesh = plsc.VectorSubcoreMesh(core_axis_name="c", subcore_axis_name="s")

    @functools.partial(
        pl.kernel,
        mesh=mesh,
        out_type=jax.ShapeDtypeStruct((N,), jnp.float32),
        scratch_types=[
            pltpu.VMEM_SHARED((VOCAB_N,), jnp.float32),
            pltpu.VMEM((CHUNK,), jnp.int32),
            pltpu.VMEM((CHUNK,), jnp.int32),
            pltpu.VMEM((CHUNK,), jnp.float32),
            pltpu.VMEM((CHUNK,), jnp.float32),
            pltpu.SemaphoreType.DMA,
            pltpu.SemaphoreType.DMA,
            pltpu.SemaphoreType.DMA,
            pltpu.SemaphoreType.DMA,
            pltpu.SemaphoreType.DMA,
            pltpu.SemaphoreType.DMA,
        ],
    )
    def gather_kernel(tok_hbm, tab_hbm, out_hbm, tab_sp, idx0, idx1,
                      val0, val1, si0, si1, so0, so1, sga, sgb):
        s = lax.axis_index("s")
        wid = s * 2 + lax.axis_index("c")
        base = wid * BPW
        idx = (idx0, idx1)
        val = (val0, val1)
        sem_i = (si0, si1)
        sem_o = (so0, so1)
        sem_g = (sga, sgb)

        pltpu.async_copy(tok_hbm.at[pl.ds(base, CHUNK)], idx0, si0)

        def seg_off(r):
            return (r * 16 + s) * SEG

        def hop1(r):
            pltpu.async_copy(tab_hbm.at[pl.ds(seg_off(r), SEG)],
                             val[r % 2].at[pl.ds(0, SEG)], sem_g[r % 2])

        def hop1_wait(r):
            pltpu.make_async_copy(tab_hbm.at[pl.ds(seg_off(r), SEG)],
                                  val[r % 2].at[pl.ds(0, SEG)],
                                  sem_g[r % 2]).wait()

        def hop2(r):
            pltpu.async_copy(val[r % 2].at[pl.ds(0, SEG)],
                             tab_sp.at[pl.ds(seg_off(r), SEG)],
                             sem_g[r % 2])

        def hop2_wait(r):
            pltpu.make_async_copy(val[r % 2].at[pl.ds(0, SEG)],
                                  tab_sp.at[pl.ds(seg_off(r), SEG)],
                                  sem_g[r % 2]).wait()

        hop1(0)
        for r in range(NROUND):
            if r == NROUND - 1:
                @pl.when(r * 16 + s < NSEG)
                def _():
                    hop1_wait(r)
                    hop2(r)
            else:
                hop1_wait(r)
                hop2(r)
                nxt = r + 1
                if nxt == NROUND - 1:
                    @pl.when(nxt * 16 + s < NSEG)
                    def _():
                        if nxt >= 2:
                            hop2_wait(nxt - 2)
                        hop1(nxt)
                else:
                    if nxt >= 2:
                        hop2_wait(nxt - 2)
                    hop1(nxt)
        @pl.when((NROUND - 1) * 16 + s < NSEG)
        def _():
            hop2_wait(NROUND - 2)
            hop2_wait(NROUND - 1)

        @pl.when((NROUND - 1) * 16 + s >= NSEG)
        def _():
            hop2_wait(NROUND - 3)
            hop2_wait(NROUND - 2)

        plsc.subcore_barrier()

        for i in range(NCHUNK):
            b = i % 2
            if i + 1 < NCHUNK:
                pltpu.async_copy(
                    tok_hbm.at[pl.ds(base + (i + 1) * CHUNK, CHUNK)],
                    idx[1 - b], sem_i[1 - b])
            if i >= 2:
                pltpu.make_async_copy(
                    val[b], out_hbm.at[pl.ds(base + (i - 2) * CHUNK, CHUNK)],
                    sem_o[b]).wait()
            pltpu.make_async_copy(
                tok_hbm.at[pl.ds(base + i * CHUNK, CHUNK)], idx[b],
                sem_i[b]).wait()
            pltpu.async_copy(tab_sp.at[idx[b]], val[b], sga).wait()
            pltpu.async_copy(
                val[b], out_hbm.at[pl.ds(base + i * CHUNK, CHUNK)], sem_o[b])
        for i in range(NCHUNK - 2, NCHUNK):
            b = i % 2
            pltpu.make_async_copy(
                val[b], out_hbm.at[pl.ds(base + i * CHUNK, CHUNK)],
                sem_o[b]).wait()

    return gather_kernel


_GATHER = _make_kernel()


def kernel(tokens, vocab_table):
    flat = tokens.reshape(N)
    out = _GATHER(flat, vocab_table)
    return out.reshape(BATCH, HIST)


# speedup vs baseline: 1.2356x; 1.0061x over previous
import functools

import jax
import jax.numpy as jnp
from jax import lax
from jax.experimental import pallas as pl
from jax.experimental.pallas import tpu as pltpu
from jax.experimental.pallas import tpu_sc as plsc

BATCH = 16384
HIST = 200
N = BATCH * HIST
VOCAB_N = 1_000_000
NUM_WORKERS = 32
BPW = N // NUM_WORKERS
CHUNK = 12_800
NCHUNK = BPW // CHUNK
SEG = 10_400
NROUND = 6
TAIL_OFF = NROUND * 16 * SEG
TAIL_SEG = 200


def _make_kernel():
    mesh = plsc.VectorSubcoreMesh(core_axis_name="c", subcore_axis_name="s")

    @functools.partial(
        pl.kernel,
        mesh=mesh,
        out_type=jax.ShapeDtypeStruct((N,), jnp.float32),
        scratch_types=[
            pltpu.VMEM_SHARED((VOCAB_N,), jnp.float32),
            pltpu.VMEM((CHUNK,), jnp.int32),
            pltpu.VMEM((CHUNK,), jnp.int32),
            pltpu.VMEM((CHUNK,), jnp.float32),
            pltpu.VMEM((CHUNK,), jnp.float32),
            pltpu.SemaphoreType.DMA,
            pltpu.SemaphoreType.DMA,
            pltpu.SemaphoreType.DMA,
            pltpu.SemaphoreType.DMA,
            pltpu.SemaphoreType.DMA,
            pltpu.SemaphoreType.DMA,
        ],
    )
    def gather_kernel(tok_hbm, tab_hbm, out_hbm, tab_sp, idx0, idx1,
                      val0, val1, si0, si1, so0, so1, sga, sgb):
        s = lax.axis_index("s")
        wid = s * 2 + lax.axis_index("c")
        base = wid * BPW
        idx = (idx0, idx1)
        val = (val0, val1)
        sem_i = (si0, si1)
        sem_o = (so0, so1)
        sem_g = (sga, sgb)

        pltpu.async_copy(tok_hbm.at[pl.ds(base, CHUNK)], idx0, si0)

        def slices(r):
            if r < NROUND:
                off, ln = (r * 16 + s) * SEG, SEG
            else:
                off, ln = TAIL_OFF + s * TAIL_SEG, TAIL_SEG
            return (tab_hbm.at[pl.ds(off, ln)],
                    val[r % 2].at[pl.ds(0, ln)],
                    tab_sp.at[pl.ds(off, ln)],
                    sem_g[r % 2])

        def hop1(r):
            src, buf, _, sem = slices(r)
            pltpu.async_copy(src, buf, sem)

        def hop1_wait(r):
            src, buf, _, sem = slices(r)
            pltpu.make_async_copy(src, buf, sem).wait()

        def hop2(r):
            _, buf, dst, sem = slices(r)
            pltpu.async_copy(buf, dst, sem)

        def hop2_wait(r):
            _, buf, dst, sem = slices(r)
            pltpu.make_async_copy(buf, dst, sem).wait()

        hop1(0)
        for r in range(NROUND):
            hop1_wait(r)
            hop2(r)
            nxt = r + 1
            if nxt < NROUND:
                if nxt >= 2:
                    hop2_wait(nxt - 2)
                hop1(nxt)
            else:
                @pl.when(s < 8)
                def _():
                    hop2_wait(nxt - 2)
                    hop1(nxt)
        @pl.when(s < 8)
        def _():
            hop1_wait(NROUND)
            hop2(NROUND)
            hop2_wait(NROUND - 1)
            hop2_wait(NROUND)

        @pl.when(s >= 8)
        def _():
            hop2_wait(NROUND - 2)
            hop2_wait(NROUND - 1)

        plsc.subcore_barrier()

        for i in range(NCHUNK):
            b = i % 2
            if i + 1 < NCHUNK:
                pltpu.async_copy(
                    tok_hbm.at[pl.ds(base + (i + 1) * CHUNK, CHUNK)],
                    idx[1 - b], sem_i[1 - b])
            if i >= 2:
                pltpu.make_async_copy(
                    val[b], out_hbm.at[pl.ds(base + (i - 2) * CHUNK, CHUNK)],
                    sem_o[b]).wait()
            pltpu.make_async_copy(
                tok_hbm.at[pl.ds(base + i * CHUNK, CHUNK)], idx[b],
                sem_i[b]).wait()
            pltpu.async_copy(tab_sp.at[idx[b]], val[b], sga).wait()
            pltpu.async_copy(
                val[b], out_hbm.at[pl.ds(base + i * CHUNK, CHUNK)], sem_o[b])
        for i in range(NCHUNK - 2, NCHUNK):
            b = i % 2
            pltpu.make_async_copy(
                val[b], out_hbm.at[pl.ds(base + i * CHUNK, CHUNK)],
                sem_o[b]).wait()

    return gather_kernel


_GATHER = _make_kernel()


def kernel(tokens, vocab_table):
    flat = tokens.reshape(N)
    out = _GATHER(flat, vocab_table)
    return out.reshape(BATCH, HIST)
